# Initial kernel scaffold; baseline (speedup 1.0000x reference)
import jax
import jax.numpy as jnp
from jax.experimental import pallas as pl


def kernel(node_x, nodes_map, edge_batch, edges_map, node_batch, We, be, gE, bE, Wn, bn, gN, bN, Wc, bc):
    raise NotImplementedError("write your pallas kernel here")


# speedup vs baseline: 4.5071x; 4.5071x over previous
import jax
import jax.numpy as jnp
from jax import lax
from jax.experimental import pallas as pl
from jax.experimental.pallas import tpu as pltpu
from jax.experimental.pallas import tpu_sc as plsc

_N = 10000
_M = 5000
_I = 320000
_D = 128
_C = 40
_NC = 2
_NS = 16
_NW = _NC * _NS
_CH = _I // _NW
_B = 80
_NCHUNK = _CH // _B
_MP = 5120
_NP = 10240
_CW = 16


def _seg_sum(x, gidx, seg, SP, with_cnt):
    rpt = SP // _NS
    mesh = plsc.VectorSubcoreMesh(core_axis_name="c", subcore_axis_name="s",
                                  num_cores=_NC, num_subcores=_NS)

    if with_cnt:
        out_type = (
            jax.ShapeDtypeStruct((_NC, SP, _D), jnp.float32),
            jax.ShapeDtypeStruct((_NC, SP, _CW), jnp.float32),
        )
    else:
        out_type = jax.ShapeDtypeStruct((_NC, SP, _D), jnp.float32)

    scratch = [
        pltpu.VMEM_SHARED((SP, _D), jnp.float32),
        pltpu.VMEM((_B, _D), jnp.float32),
        pltpu.VMEM((_B,), jnp.int32),
        pltpu.VMEM((_B,), jnp.int32),
        pltpu.SemaphoreType.DMA,
    ]
    if with_cnt:
        scratch.append(pltpu.VMEM_SHARED((SP, _CW), jnp.float32))
        scratch.append(pltpu.VMEM((_B, _CW), jnp.float32))
        scratch.append(pltpu.VMEM((_B, _CW), jnp.float32))

    def body(x_h, gi_h, sg_h, zr_h, *rest):
        if with_cnt:
            (z16_h, on_h, acc_o, cnt_o,
             acc_s, rows_v, gi_v, sg_v, sem, cnt_s, ones_v, zc_v) = rest
        else:
            (acc_o, acc_s, rows_v, gi_v, sg_v, sem) = rest
        cid = lax.axis_index("c")
        sid = lax.axis_index("s")
        wid = cid * _NS + sid
        r0 = sid * rpt
        pltpu.sync_copy(zr_h, rows_v)
        if with_cnt:
            pltpu.sync_copy(z16_h, zc_v)
            pltpu.sync_copy(on_h, ones_v)
        for j in range(rpt // _B):
            pltpu.sync_copy(rows_v, acc_s.at[pl.ds(r0 + j * _B, _B)])
            if with_cnt:
                pltpu.sync_copy(zc_v, cnt_s.at[pl.ds(r0 + j * _B, _B)])
        plsc.subcore_barrier()
        base0 = wid * _CH

        def chunk(k, carry):
            b = base0 + k * _B
            pltpu.sync_copy(gi_h.at[pl.ds(b, _B)], gi_v)
            pltpu.sync_copy(sg_h.at[pl.ds(b, _B)], sg_v)
            pltpu.async_copy(x_h.at[gi_v], rows_v, sem).wait()
            pltpu.sync_copy(rows_v, acc_s.at[sg_v], add=True)
            if with_cnt:
                pltpu.sync_copy(ones_v, cnt_s.at[sg_v], add=True)
            return carry

        lax.fori_loop(0, _NCHUNK, chunk, 0)
        plsc.subcore_barrier()
        for j in range(rpt // _B):
            pltpu.sync_copy(acc_s.at[pl.ds(r0 + j * _B, _B)], rows_v)
            pltpu.sync_copy(rows_v, acc_o.at[cid, pl.ds(r0 + j * _B, _B)])
            if with_cnt:
                pltpu.sync_copy(cnt_s.at[pl.ds(r0 + j * _B, _B)], zc_v)
                pltpu.sync_copy(zc_v, cnt_o.at[cid, pl.ds(r0 + j * _B, _B)])

    f = pl.kernel(body, out_type=out_type, mesh=mesh,
                  scratch_types=tuple(scratch),
                  compiler_params=pltpu.CompilerParams(
                      use_tc_tiling_on_sc=False))
    zrow = jnp.zeros((_B, _D), jnp.float32)
    if with_cnt:
        z16 = jnp.zeros((_B, _CW), jnp.float32)
        ones = jnp.ones((_B, _CW), jnp.float32)
        return f(x, gidx, seg, zrow, z16, ones)
    return f(x, gidx, seg, zrow)


def _dense_update(acc2, cnt2, W, b, g, bt, BLK=512):
    SP = acc2.shape[1]

    def body(a_r, c_r, w_r, b_r, g_r, t_r, o_r):
        a = a_r[0] + a_r[1]
        c = c_r[0, :, 0:1] + c_r[1, :, 0:1]
        m = a / jnp.maximum(c, 1.0)
        z = jnp.dot(m, w_r[...], preferred_element_type=jnp.float32) + b_r[...]
        mu = jnp.mean(z, axis=-1, keepdims=True)
        var = jnp.mean((z - mu) ** 2, axis=-1, keepdims=True)
        y = (z - mu) * lax.rsqrt(var + 1e-5) * g_r[...] + t_r[...]
        o_r[...] = jnp.maximum(y, 0.0)

    return pl.pallas_call(
        body,
        grid=(SP // BLK,),
        in_specs=[
            pl.BlockSpec((_NC, BLK, _D), lambda i: (0, i, 0)),
            pl.BlockSpec((_NC, BLK, _CW), lambda i: (0, i, 0)),
            pl.BlockSpec((_D, _D), lambda i: (0, 0)),
            pl.BlockSpec((1, _D), lambda i: (0, 0)),
            pl.BlockSpec((1, _D), lambda i: (0, 0)),
            pl.BlockSpec((1, _D), lambda i: (0, 0)),
        ],
        out_specs=pl.BlockSpec((BLK, _D), lambda i: (i, 0)),
        out_shape=jax.ShapeDtypeStruct((SP, _D), jnp.float32),
    )(acc2, cnt2, W, b.reshape(1, _D), g.reshape(1, _D), bt.reshape(1, _D))


def _final_update(acc2, cnt2, W, b, g, bt, Wc, bc, BLK=512):
    SP = acc2.shape[1]
    Wcp = jnp.zeros((_D, _D), jnp.float32).at[:, :_C].set(Wc)
    bcp = jnp.full((1, _D), -1e30, jnp.float32).at[0, :_C].set(bc)

    def body(a_r, c_r, w_r, b_r, g_r, t_r, wc_r, bc_r, o_r):
        a = a_r[0] + a_r[1]
        c = c_r[0, :, 0:1] + c_r[1, :, 0:1]
        m = a / jnp.maximum(c, 1.0)
        z = jnp.dot(m, w_r[...], preferred_element_type=jnp.float32) + b_r[...]
        mu = jnp.mean(z, axis=-1, keepdims=True)
        var = jnp.mean((z - mu) ** 2, axis=-1, keepdims=True)
        h = jnp.maximum((z - mu) * lax.rsqrt(var + 1e-5) * g_r[...] + t_r[...],
                        0.0)
        lg = jnp.dot(h, wc_r[...], preferred_element_type=jnp.float32) + bc_r[...]
        mx = jnp.max(lg, axis=-1, keepdims=True)
        lse = mx + jnp.log(jnp.sum(jnp.exp(lg - mx), axis=-1, keepdims=True))
        o_r[...] = lg - lse

    return pl.pallas_call(
        body,
        grid=(SP // BLK,),
        in_specs=[
            pl.BlockSpec((_NC, BLK, _D), lambda i: (0, i, 0)),
            pl.BlockSpec((_NC, BLK, _CW), lambda i: (0, i, 0)),
            pl.BlockSpec((_D, _D), lambda i: (0, 0)),
            pl.BlockSpec((1, _D), lambda i: (0, 0)),
            pl.BlockSpec((1, _D), lambda i: (0, 0)),
            pl.BlockSpec((1, _D), lambda i: (0, 0)),
            pl.BlockSpec((_D, _D), lambda i: (0, 0)),
            pl.BlockSpec((1, _D), lambda i: (0, 0)),
        ],
        out_specs=pl.BlockSpec((BLK, _D), lambda i: (i, 0)),
        out_shape=jax.ShapeDtypeStruct((SP, _D), jnp.float32),
    )(acc2, cnt2, W, b.reshape(1, _D), g.reshape(1, _D), bt.reshape(1, _D),
      Wcp, bcp)


def kernel(node_x, nodes_map, edge_batch, edges_map, node_batch,
           We, be, gE, bE, Wn, bn, gN, bN, Wc, bc):
    nm = nodes_map.astype(jnp.int32)
    em = edges_map.astype(jnp.int32)
    eb = edge_batch.astype(jnp.int32)
    nb = node_batch.astype(jnp.int32)

    acc_e, cnt_e = _seg_sum(node_x, nm, eb, _MP, True)
    edge_x = _dense_update(acc_e, cnt_e, We[0], be[0], gE[0], bE[0])
    acc_n, cnt_n = _seg_sum(edge_x, em, nb, _NP, True)
    x1 = _dense_update(acc_n, cnt_n, Wn[0], bn[0], gN[0], bN[0])
    acc_e2 = _seg_sum(x1, nm, eb, _MP, False)
    edge_x2 = _dense_update(acc_e2, cnt_e, We[1], be[1], gE[1], bE[1])
    acc_n2 = _seg_sum(edge_x2, em, nb, _NP, False)
    out = _final_update(acc_n2, cnt_n, Wn[1], bn[1], gN[1], bN[1], Wc, bc)
    return out[:_N, :_C]


# speedup vs baseline: 8.0530x; 1.7867x over previous
import jax
import jax.numpy as jnp
from jax import lax
from jax.experimental import pallas as pl
from jax.experimental.pallas import tpu as pltpu
from jax.experimental.pallas import tpu_sc as plsc

_N = 10000
_M = 5000
_I = 320000
_D = 128
_C = 40
_NC = 2
_NS = 16
_NW = _NC * _NS
_CH = _I // _NW
_B = 80
_NCHUNK = _CH // _B
_MP = 5120
_NP = 10240
_CW = 16


def _seg_sum(x, gidx, seg, SP, with_cnt, B):
    rpt = SP // _NS
    nchunk = _CH // B
    ZB = 80
    mesh = plsc.VectorSubcoreMesh(core_axis_name="c", subcore_axis_name="s",
                                  num_cores=_NC, num_subcores=_NS)

    if with_cnt:
        out_type = (
            jax.ShapeDtypeStruct((_NC, SP, _D), jnp.float32),
            jax.ShapeDtypeStruct((_NC, SP, _CW), jnp.float32),
        )
    else:
        out_type = jax.ShapeDtypeStruct((_NC, SP, _D), jnp.float32)

    scratch = [
        pltpu.VMEM_SHARED((SP, _D), jnp.float32),
        pltpu.VMEM((B, _D), jnp.float32),
        pltpu.VMEM((B, _D), jnp.float32),
        pltpu.VMEM((B,), jnp.int32),
        pltpu.VMEM((B,), jnp.int32),
        pltpu.VMEM((B,), jnp.int32),
        pltpu.VMEM((B,), jnp.int32),
        pltpu.SemaphoreType.DMA,
        pltpu.SemaphoreType.DMA,
    ]
    if with_cnt:
        scratch.append(pltpu.VMEM_SHARED((SP, _CW), jnp.float32))
        scratch.append(pltpu.VMEM((B, _CW), jnp.float32))
        scratch.append(pltpu.VMEM((ZB, _CW), jnp.float32))

    def body(x_h, gi_h, sg_h, zr_h, *rest):
        if with_cnt:
            (z16_h, on_h, acc_o, cnt_o, acc_s,
             rows0, rows1, gi0, gi1, sg0, sg1, sem0, sem1,
             cnt_s, ones_v, zc_v) = rest
        else:
            (acc_o, acc_s,
             rows0, rows1, gi0, gi1, sg0, sg1, sem0, sem1) = rest
        rows = (rows0, rows1)
        gi = (gi0, gi1)
        sg = (sg0, sg1)
        sem = (sem0, sem1)
        cid = lax.axis_index("c")
        sid = lax.axis_index("s")
        wid = cid * _NS + sid
        r0 = sid * rpt
        base0 = wid * _CH

        def load_idx(n, q):
            b = base0 + n * B
            pltpu.sync_copy(gi_h.at[pl.ds(b, B)], gi[q])
            pltpu.sync_copy(sg_h.at[pl.ds(b, B)], sg[q])

        def start_gather(q):
            pltpu.async_copy(x_h.at[gi[q]], rows[q], sem[q])

        def wait_gather(p):
            pltpu.make_async_copy(x_h.at[gi[p]], rows[p], sem[p]).wait()

        def scatter(p):
            pltpu.sync_copy(rows[p], acc_s.at[sg[p]], add=True)
            if with_cnt:
                pltpu.sync_copy(ones_v, cnt_s.at[sg[p]], add=True)

        load_idx(0, 0)
        start_gather(0)
        pltpu.sync_copy(zr_h, rows1.at[pl.ds(0, ZB)])
        if with_cnt:
            pltpu.sync_copy(z16_h, zc_v)
            pltpu.sync_copy(on_h, ones_v)
        for j in range(rpt // ZB):
            pltpu.sync_copy(rows1.at[pl.ds(0, ZB)],
                            acc_s.at[pl.ds(r0 + j * ZB, ZB)])
            if with_cnt:
                pltpu.sync_copy(zc_v, cnt_s.at[pl.ds(r0 + j * ZB, ZB)])
        plsc.subcore_barrier()

        def half(n, p):
            q = 1 - p
            load_idx(n + 1, q)
            wait_gather(p)
            start_gather(q)
            scatter(p)

        npairs = (nchunk - 1) // 2

        def pair(j, carry):
            half(2 * j, 0)
            half(2 * j + 1, 1)
            return carry

        lax.fori_loop(0, npairs, pair, 0)
        for n in range(2 * npairs, nchunk):
            p = n % 2
            wait_gather(p)
            if n + 1 < nchunk:
                load_idx(n + 1, 1 - p)
                start_gather(1 - p)
            scatter(p)
        plsc.subcore_barrier()
        for j in range(rpt // ZB):
            pltpu.sync_copy(acc_s.at[pl.ds(r0 + j * ZB, ZB)],
                            rows0.at[pl.ds(0, ZB)])
            pltpu.sync_copy(rows0.at[pl.ds(0, ZB)],
                            acc_o.at[cid, pl.ds(r0 + j * ZB, ZB)])
            if with_cnt:
                pltpu.sync_copy(cnt_s.at[pl.ds(r0 + j * ZB, ZB)], zc_v)
                pltpu.sync_copy(zc_v, cnt_o.at[cid, pl.ds(r0 + j * ZB, ZB)])

    f = pl.kernel(body, out_type=out_type, mesh=mesh,
                  scratch_types=tuple(scratch),
                  compiler_params=pltpu.CompilerParams(
                      use_tc_tiling_on_sc=False))
    zrow = jnp.zeros((ZB, _D), jnp.float32)
    if with_cnt:
        z16 = jnp.zeros((ZB, _CW), jnp.float32)
        ones = jnp.ones((B, _CW), jnp.float32)
        return f(x, gidx, seg, zrow, z16, ones)
    return f(x, gidx, seg, zrow)


def _dense_update(acc2, cnt2, W, b, g, bt, BLK=512):
    SP = acc2.shape[1]

    def body(a_r, c_r, w_r, b_r, g_r, t_r, o_r):
        a = a_r[0] + a_r[1]
        c = c_r[0, :, 0:1] + c_r[1, :, 0:1]
        m = a / jnp.maximum(c, 1.0)
        z = jnp.dot(m, w_r[...], preferred_element_type=jnp.float32) + b_r[...]
        mu = jnp.mean(z, axis=-1, keepdims=True)
        var = jnp.mean((z - mu) ** 2, axis=-1, keepdims=True)
        y = (z - mu) * lax.rsqrt(var + 1e-5) * g_r[...] + t_r[...]
        o_r[...] = jnp.maximum(y, 0.0)

    return pl.pallas_call(
        body,
        grid=(SP // BLK,),
        in_specs=[
            pl.BlockSpec((_NC, BLK, _D), lambda i: (0, i, 0)),
            pl.BlockSpec((_NC, BLK, _CW), lambda i: (0, i, 0)),
            pl.BlockSpec((_D, _D), lambda i: (0, 0)),
            pl.BlockSpec((1, _D), lambda i: (0, 0)),
            pl.BlockSpec((1, _D), lambda i: (0, 0)),
            pl.BlockSpec((1, _D), lambda i: (0, 0)),
        ],
        out_specs=pl.BlockSpec((BLK, _D), lambda i: (i, 0)),
        out_shape=jax.ShapeDtypeStruct((SP, _D), jnp.float32),
    )(acc2, cnt2, W, b.reshape(1, _D), g.reshape(1, _D), bt.reshape(1, _D))


def _final_update(acc2, cnt2, W, b, g, bt, Wc, bc, BLK=512):
    SP = acc2.shape[1]
    Wcp = jnp.zeros((_D, _D), jnp.float32).at[:, :_C].set(Wc)
    bcp = jnp.full((1, _D), -1e30, jnp.float32).at[0, :_C].set(bc)

    def body(a_r, c_r, w_r, b_r, g_r, t_r, wc_r, bc_r, o_r):
        a = a_r[0] + a_r[1]
        c = c_r[0, :, 0:1] + c_r[1, :, 0:1]
        m = a / jnp.maximum(c, 1.0)
        z = jnp.dot(m, w_r[...], preferred_element_type=jnp.float32) + b_r[...]
        mu = jnp.mean(z, axis=-1, keepdims=True)
        var = jnp.mean((z - mu) ** 2, axis=-1, keepdims=True)
        h = jnp.maximum((z - mu) * lax.rsqrt(var + 1e-5) * g_r[...] + t_r[...],
                        0.0)
        lg = jnp.dot(h, wc_r[...], preferred_element_type=jnp.float32) + bc_r[...]
        mx = jnp.max(lg, axis=-1, keepdims=True)
        lse = mx + jnp.log(jnp.sum(jnp.exp(lg - mx), axis=-1, keepdims=True))
        o_r[...] = lg - lse

    return pl.pallas_call(
        body,
        grid=(SP // BLK,),
        in_specs=[
            pl.BlockSpec((_NC, BLK, _D), lambda i: (0, i, 0)),
            pl.BlockSpec((_NC, BLK, _CW), lambda i: (0, i, 0)),
            pl.BlockSpec((_D, _D), lambda i: (0, 0)),
            pl.BlockSpec((1, _D), lambda i: (0, 0)),
            pl.BlockSpec((1, _D), lambda i: (0, 0)),
            pl.BlockSpec((1, _D), lambda i: (0, 0)),
            pl.BlockSpec((_D, _D), lambda i: (0, 0)),
            pl.BlockSpec((1, _D), lambda i: (0, 0)),
        ],
        out_specs=pl.BlockSpec((BLK, _D), lambda i: (i, 0)),
        out_shape=jax.ShapeDtypeStruct((SP, _D), jnp.float32),
    )(acc2, cnt2, W, b.reshape(1, _D), g.reshape(1, _D), bt.reshape(1, _D),
      Wcp, bcp)


def kernel(node_x, nodes_map, edge_batch, edges_map, node_batch,
           We, be, gE, bE, Wn, bn, gN, bN, Wc, bc):
    nm = nodes_map.astype(jnp.int32)
    em = edges_map.astype(jnp.int32)
    eb = edge_batch.astype(jnp.int32)
    nb = node_batch.astype(jnp.int32)

    acc_e, cnt_e = _seg_sum(node_x, nm, eb, _MP, True, 200)
    edge_x = _dense_update(acc_e, cnt_e, We[0], be[0], gE[0], bE[0])
    acc_n, cnt_n = _seg_sum(edge_x, em, nb, _NP, True, 80)
    x1 = _dense_update(acc_n, cnt_n, Wn[0], bn[0], gN[0], bN[0])
    acc_e2 = _seg_sum(x1, nm, eb, _MP, False, 200)
    edge_x2 = _dense_update(acc_e2, cnt_e, We[1], be[1], gE[1], bE[1])
    acc_n2 = _seg_sum(edge_x2, em, nb, _NP, False, 80)
    out = _final_update(acc_n2, cnt_n, Wn[1], bn[1], gN[1], bN[1], Wc, bc)
    return out[:_N, :_C]


# speedup vs baseline: 8.1212x; 1.0085x over previous
import jax
import jax.numpy as jnp
from jax import lax
from jax.experimental import pallas as pl
from jax.experimental.pallas import tpu as pltpu
from jax.experimental.pallas import tpu_sc as plsc

_N = 10000
_M = 5000
_I = 320000
_D = 128
_C = 40
_NC = 2
_NS = 16
_NW = _NC * _NS
_CH = _I // _NW
_B = 80
_NCHUNK = _CH // _B
_MP = 5120
_NP = 10240
_CW = 16


def _seg_sum(x, gidx, seg, SP, with_cnt, B):
    rpt = SP // _NS
    nchunk = _CH // B
    ZB = 80
    mesh = plsc.VectorSubcoreMesh(core_axis_name="c", subcore_axis_name="s",
                                  num_cores=_NC, num_subcores=_NS)

    if with_cnt:
        out_type = (
            jax.ShapeDtypeStruct((_NC, SP, _D), jnp.float32),
            jax.ShapeDtypeStruct((_NC, SP, _CW), jnp.float32),
        )
    else:
        out_type = jax.ShapeDtypeStruct((_NC, SP, _D), jnp.float32)

    scratch = [
        pltpu.VMEM_SHARED((SP, _D), jnp.float32),
        pltpu.VMEM((B, _D), jnp.float32),
        pltpu.VMEM((B, _D), jnp.float32),
        pltpu.VMEM((B,), jnp.int32),
        pltpu.VMEM((B,), jnp.int32),
        pltpu.VMEM((B,), jnp.int32),
        pltpu.VMEM((B,), jnp.int32),
        pltpu.SemaphoreType.DMA,
        pltpu.SemaphoreType.DMA,
    ]
    if with_cnt:
        scratch.append(pltpu.VMEM_SHARED((SP, _CW), jnp.float32))
        scratch.append(pltpu.VMEM((B, _CW), jnp.float32))
        scratch.append(pltpu.VMEM((ZB, _CW), jnp.float32))

    def body(x_h, gi_h, sg_h, zr_h, *rest):
        if with_cnt:
            (z16_h, on_h, acc_o, cnt_o, acc_s,
             rows0, rows1, gi0, gi1, sg0, sg1, sem0, sem1,
             cnt_s, ones_v, zc_v) = rest
        else:
            (acc_o, acc_s,
             rows0, rows1, gi0, gi1, sg0, sg1, sem0, sem1) = rest
        rows = (rows0, rows1)
        gi = (gi0, gi1)
        sg = (sg0, sg1)
        sem = (sem0, sem1)
        cid = lax.axis_index("c")
        sid = lax.axis_index("s")
        wid = cid * _NS + sid
        r0 = sid * rpt
        base0 = wid * _CH

        def load_idx(n, q):
            b = base0 + n * B
            pltpu.sync_copy(gi_h.at[pl.ds(b, B)], gi[q])
            pltpu.sync_copy(sg_h.at[pl.ds(b, B)], sg[q])

        def start_gather(q):
            pltpu.async_copy(x_h.at[gi[q]], rows[q], sem[q])

        def wait_gather(p):
            pltpu.make_async_copy(x_h.at[gi[p]], rows[p], sem[p]).wait()

        def scatter(p):
            pltpu.sync_copy(rows[p], acc_s.at[sg[p]], add=True)
            if with_cnt:
                pltpu.sync_copy(ones_v, cnt_s.at[sg[p]], add=True)

        load_idx(0, 0)
        start_gather(0)
        pltpu.sync_copy(zr_h, rows1.at[pl.ds(0, ZB)])
        if with_cnt:
            pltpu.sync_copy(z16_h, zc_v)
            pltpu.sync_copy(on_h, ones_v)
        for j in range(rpt // ZB):
            pltpu.sync_copy(rows1.at[pl.ds(0, ZB)],
                            acc_s.at[pl.ds(r0 + j * ZB, ZB)])
            if with_cnt:
                pltpu.sync_copy(zc_v, cnt_s.at[pl.ds(r0 + j * ZB, ZB)])
        plsc.subcore_barrier()

        def half(n, p):
            q = 1 - p
            load_idx(n + 1, q)
            wait_gather(p)
            start_gather(q)
            scatter(p)

        npairs = (nchunk - 1) // 2

        def pair(j, carry):
            half(2 * j, 0)
            half(2 * j + 1, 1)
            return carry

        lax.fori_loop(0, npairs, pair, 0)
        for n in range(2 * npairs, nchunk):
            p = n % 2
            wait_gather(p)
            if n + 1 < nchunk:
                load_idx(n + 1, 1 - p)
                start_gather(1 - p)
            scatter(p)
        plsc.subcore_barrier()
        nwb = rpt // ZB
        for j in range(nwb):
            p = j % 2
            if j >= 2:
                pltpu.make_async_copy(
                    rows[p].at[pl.ds(0, ZB)],
                    acc_o.at[cid, pl.ds(r0 + (j - 2) * ZB, ZB)],
                    sem[p]).wait()
            pltpu.sync_copy(acc_s.at[pl.ds(r0 + j * ZB, ZB)],
                            rows[p].at[pl.ds(0, ZB)])
            pltpu.async_copy(rows[p].at[pl.ds(0, ZB)],
                             acc_o.at[cid, pl.ds(r0 + j * ZB, ZB)], sem[p])
        for j in range(max(nwb - 2, 0), nwb):
            p = j % 2
            pltpu.make_async_copy(
                rows[p].at[pl.ds(0, ZB)],
                acc_o.at[cid, pl.ds(r0 + j * ZB, ZB)], sem[p]).wait()
        if with_cnt:
            cbuf = (zc_v, ones_v)
            for j in range(nwb):
                p = j % 2
                if j >= 2:
                    pltpu.make_async_copy(
                        cbuf[p].at[pl.ds(0, ZB)],
                        cnt_o.at[cid, pl.ds(r0 + (j - 2) * ZB, ZB)],
                        sem[p]).wait()
                pltpu.sync_copy(cnt_s.at[pl.ds(r0 + j * ZB, ZB)],
                                cbuf[p].at[pl.ds(0, ZB)])
                pltpu.async_copy(cbuf[p].at[pl.ds(0, ZB)],
                                 cnt_o.at[cid, pl.ds(r0 + j * ZB, ZB)],
                                 sem[p])
            for j in range(max(nwb - 2, 0), nwb):
                p = j % 2
                pltpu.make_async_copy(
                    cbuf[p].at[pl.ds(0, ZB)],
                    cnt_o.at[cid, pl.ds(r0 + j * ZB, ZB)], sem[p]).wait()

    f = pl.kernel(body, out_type=out_type, mesh=mesh,
                  scratch_types=tuple(scratch),
                  compiler_params=pltpu.CompilerParams(
                      use_tc_tiling_on_sc=False))
    zrow = jnp.zeros((ZB, _D), jnp.float32)
    if with_cnt:
        z16 = jnp.zeros((ZB, _CW), jnp.float32)
        ones = jnp.ones((B, _CW), jnp.float32)
        return f(x, gidx, seg, zrow, z16, ones)
    return f(x, gidx, seg, zrow)


def _dense_update(acc2, cnt2, W, b, g, bt, BLK=512):
    SP = acc2.shape[1]

    def body(a_r, c_r, w_r, b_r, g_r, t_r, o_r):
        a = a_r[0] + a_r[1]
        c = c_r[0, :, 0:1] + c_r[1, :, 0:1]
        m = a / jnp.maximum(c, 1.0)
        z = jnp.dot(m, w_r[...], preferred_element_type=jnp.float32) + b_r[...]
        mu = jnp.mean(z, axis=-1, keepdims=True)
        var = jnp.mean((z - mu) ** 2, axis=-1, keepdims=True)
        y = (z - mu) * lax.rsqrt(var + 1e-5) * g_r[...] + t_r[...]
        o_r[...] = jnp.maximum(y, 0.0)

    return pl.pallas_call(
        body,
        grid=(SP // BLK,),
        in_specs=[
            pl.BlockSpec((_NC, BLK, _D), lambda i: (0, i, 0)),
            pl.BlockSpec((_NC, BLK, _CW), lambda i: (0, i, 0)),
            pl.BlockSpec((_D, _D), lambda i: (0, 0)),
            pl.BlockSpec((1, _D), lambda i: (0, 0)),
            pl.BlockSpec((1, _D), lambda i: (0, 0)),
            pl.BlockSpec((1, _D), lambda i: (0, 0)),
        ],
        out_specs=pl.BlockSpec((BLK, _D), lambda i: (i, 0)),
        out_shape=jax.ShapeDtypeStruct((SP, _D), jnp.float32),
    )(acc2, cnt2, W, b.reshape(1, _D), g.reshape(1, _D), bt.reshape(1, _D))


def _final_update(acc2, cnt2, W, b, g, bt, Wc, bc, BLK=512):
    SP = acc2.shape[1]
    Wcp = jnp.zeros((_D, _D), jnp.float32).at[:, :_C].set(Wc)
    bcp = jnp.full((1, _D), -1e30, jnp.float32).at[0, :_C].set(bc)

    def body(a_r, c_r, w_r, b_r, g_r, t_r, wc_r, bc_r, o_r):
        a = a_r[0] + a_r[1]
        c = c_r[0, :, 0:1] + c_r[1, :, 0:1]
        m = a / jnp.maximum(c, 1.0)
        z = jnp.dot(m, w_r[...], preferred_element_type=jnp.float32) + b_r[...]
        mu = jnp.mean(z, axis=-1, keepdims=True)
        var = jnp.mean((z - mu) ** 2, axis=-1, keepdims=True)
        h = jnp.maximum((z - mu) * lax.rsqrt(var + 1e-5) * g_r[...] + t_r[...],
                        0.0)
        lg = jnp.dot(h, wc_r[...], preferred_element_type=jnp.float32) + bc_r[...]
        mx = jnp.max(lg, axis=-1, keepdims=True)
        lse = mx + jnp.log(jnp.sum(jnp.exp(lg - mx), axis=-1, keepdims=True))
        o_r[...] = lg - lse

    return pl.pallas_call(
        body,
        grid=(SP // BLK,),
        in_specs=[
            pl.BlockSpec((_NC, BLK, _D), lambda i: (0, i, 0)),
            pl.BlockSpec((_NC, BLK, _CW), lambda i: (0, i, 0)),
            pl.BlockSpec((_D, _D), lambda i: (0, 0)),
            pl.BlockSpec((1, _D), lambda i: (0, 0)),
            pl.BlockSpec((1, _D), lambda i: (0, 0)),
            pl.BlockSpec((1, _D), lambda i: (0, 0)),
            pl.BlockSpec((_D, _D), lambda i: (0, 0)),
            pl.BlockSpec((1, _D), lambda i: (0, 0)),
        ],
        out_specs=pl.BlockSpec((BLK, _D), lambda i: (i, 0)),
        out_shape=jax.ShapeDtypeStruct((SP, _D), jnp.float32),
    )(acc2, cnt2, W, b.reshape(1, _D), g.reshape(1, _D), bt.reshape(1, _D),
      Wcp, bcp)


def kernel(node_x, nodes_map, edge_batch, edges_map, node_batch,
           We, be, gE, bE, Wn, bn, gN, bN, Wc, bc):
    nm = nodes_map.astype(jnp.int32)
    em = edges_map.astype(jnp.int32)
    eb = edge_batch.astype(jnp.int32)
    nb = node_batch.astype(jnp.int32)

    acc_e, cnt_e = _seg_sum(node_x, nm, eb, _MP, True, 200)
    edge_x = _dense_update(acc_e, cnt_e, We[0], be[0], gE[0], bE[0])
    acc_n, cnt_n = _seg_sum(edge_x, em, nb, _NP, True, 80)
    x1 = _dense_update(acc_n, cnt_n, Wn[0], bn[0], gN[0], bN[0])
    acc_e2 = _seg_sum(x1, nm, eb, _MP, False, 200)
    edge_x2 = _dense_update(acc_e2, cnt_e, We[1], be[1], gE[1], bE[1])
    acc_n2 = _seg_sum(edge_x2, em, nb, _NP, False, 80)
    out = _final_update(acc_n2, cnt_n, Wn[1], bn[1], gN[1], bN[1], Wc, bc)
    return out[:_N, :_C]


# speedup vs baseline: 8.1950x; 1.0091x over previous
import jax
import jax.numpy as jnp
from jax import lax
from jax.experimental import pallas as pl
from jax.experimental.pallas import tpu as pltpu
from jax.experimental.pallas import tpu_sc as plsc

_N = 10000
_M = 5000
_I = 320000
_D = 128
_C = 40
_NC = 2
_NS = 16
_NW = _NC * _NS
_CH = _I // _NW
_B = 80
_NCHUNK = _CH // _B
_MP = 5120
_NP = 10240
_CW = 16


def _seg_sum(x, gidx, seg, SP, with_cnt, B):
    rpt = SP // _NS
    nchunk = _CH // B
    ZB = 80
    mesh = plsc.VectorSubcoreMesh(core_axis_name="c", subcore_axis_name="s",
                                  num_cores=_NC, num_subcores=_NS)

    if with_cnt:
        out_type = (
            jax.ShapeDtypeStruct((_NC, SP, _D), jnp.float32),
            jax.ShapeDtypeStruct((_NC, SP, _CW), jnp.float32),
        )
    else:
        out_type = jax.ShapeDtypeStruct((_NC, SP, _D), jnp.float32)

    scratch = [
        pltpu.VMEM_SHARED((SP, _D), jnp.float32),
        pltpu.VMEM((B, _D), jnp.float32),
        pltpu.VMEM((B, _D), jnp.float32),
        pltpu.VMEM((B,), jnp.int32),
        pltpu.VMEM((B,), jnp.int32),
        pltpu.VMEM((B,), jnp.int32),
        pltpu.VMEM((B,), jnp.int32),
        pltpu.SemaphoreType.DMA,
        pltpu.SemaphoreType.DMA,
        pltpu.SemaphoreType.DMA,
        pltpu.SemaphoreType.DMA,
    ]
    if with_cnt:
        scratch.append(pltpu.VMEM_SHARED((SP, _CW), jnp.float32))
        scratch.append(pltpu.VMEM((B, _CW), jnp.float32))
        scratch.append(pltpu.VMEM((ZB, _CW), jnp.float32))

    def body(x_h, gi_h, sg_h, zr_h, *rest):
        if with_cnt:
            (z16_h, on_h, acc_o, cnt_o, acc_s,
             rows0, rows1, gi0, gi1, sg0, sg1, sem0, sem1, ssem0, ssem1,
             cnt_s, ones_v, zc_v) = rest
        else:
            (acc_o, acc_s, rows0, rows1, gi0, gi1, sg0, sg1,
             sem0, sem1, ssem0, ssem1) = rest
        rows = (rows0, rows1)
        gi = (gi0, gi1)
        sg = (sg0, sg1)
        sem = (sem0, sem1)
        ssem = (ssem0, ssem1)
        cid = lax.axis_index("c")
        sid = lax.axis_index("s")
        wid = cid * _NS + sid
        r0 = sid * rpt
        base0 = wid * _CH

        def load_idx(n, q):
            b = base0 + n * B
            pltpu.sync_copy(gi_h.at[pl.ds(b, B)], gi[q])
            pltpu.sync_copy(sg_h.at[pl.ds(b, B)], sg[q])

        def start_gather(q):
            pltpu.async_copy(x_h.at[gi[q]], rows[q], sem[q])

        def wait_gather(p):
            pltpu.make_async_copy(x_h.at[gi[p]], rows[p], sem[p]).wait()

        def start_scatter(p):
            pltpu.async_copy(rows[p], acc_s.at[sg[p]], ssem[p], add=True)
            if with_cnt:
                pltpu.async_copy(ones_v, cnt_s.at[sg[p]], ssem[p], add=True)

        def wait_scatter(p):
            pltpu.make_async_copy(rows[p], acc_s.at[sg[p]], ssem[p]).wait()
            if with_cnt:
                pltpu.make_async_copy(ones_v, cnt_s.at[sg[p]],
                                      ssem[p]).wait()

        load_idx(0, 0)
        start_gather(0)
        pltpu.sync_copy(zr_h, rows1.at[pl.ds(0, ZB)])
        if with_cnt:
            pltpu.sync_copy(z16_h, zc_v)
            pltpu.sync_copy(on_h, ones_v)
        for j in range(rpt // ZB):
            pltpu.sync_copy(rows1.at[pl.ds(0, ZB)],
                            acc_s.at[pl.ds(r0 + j * ZB, ZB)])
            if with_cnt:
                pltpu.sync_copy(zc_v, cnt_s.at[pl.ds(r0 + j * ZB, ZB)])
        plsc.subcore_barrier()

        def half(n, p, first=False):
            q = 1 - p
            if not first:
                wait_scatter(q)
            load_idx(n + 1, q)
            wait_gather(p)
            start_gather(q)
            start_scatter(p)

        npairs = (nchunk - 1) // 2
        half(0, 0, first=True)
        half(1, 1)

        def pair(j, carry):
            half(2 * j, 0)
            half(2 * j + 1, 1)
            return carry

        lax.fori_loop(1, npairs, pair, 0)
        for n in range(2 * npairs, nchunk):
            p = n % 2
            if n + 1 < nchunk:
                wait_scatter(1 - p)
                load_idx(n + 1, 1 - p)
            wait_gather(p)
            if n + 1 < nchunk:
                start_gather(1 - p)
            start_scatter(p)
        wait_scatter(0)
        wait_scatter(1)
        plsc.subcore_barrier()
        nwb = rpt // ZB
        for j in range(nwb):
            p = j % 2
            if j >= 2:
                pltpu.make_async_copy(
                    rows[p].at[pl.ds(0, ZB)],
                    acc_o.at[cid, pl.ds(r0 + (j - 2) * ZB, ZB)],
                    sem[p]).wait()
            pltpu.sync_copy(acc_s.at[pl.ds(r0 + j * ZB, ZB)],
                            rows[p].at[pl.ds(0, ZB)])
            pltpu.async_copy(rows[p].at[pl.ds(0, ZB)],
                             acc_o.at[cid, pl.ds(r0 + j * ZB, ZB)], sem[p])
        for j in range(max(nwb - 2, 0), nwb):
            p = j % 2
            pltpu.make_async_copy(
                rows[p].at[pl.ds(0, ZB)],
                acc_o.at[cid, pl.ds(r0 + j * ZB, ZB)], sem[p]).wait()
        if with_cnt:
            cbuf = (zc_v, ones_v)
            for j in range(nwb):
                p = j % 2
                if j >= 2:
                    pltpu.make_async_copy(
                        cbuf[p].at[pl.ds(0, ZB)],
                        cnt_o.at[cid, pl.ds(r0 + (j - 2) * ZB, ZB)],
                        sem[p]).wait()
                pltpu.sync_copy(cnt_s.at[pl.ds(r0 + j * ZB, ZB)],
                                cbuf[p].at[pl.ds(0, ZB)])
                pltpu.async_copy(cbuf[p].at[pl.ds(0, ZB)],
                                 cnt_o.at[cid, pl.ds(r0 + j * ZB, ZB)],
                                 sem[p])
            for j in range(max(nwb - 2, 0), nwb):
                p = j % 2
                pltpu.make_async_copy(
                    cbuf[p].at[pl.ds(0, ZB)],
                    cnt_o.at[cid, pl.ds(r0 + j * ZB, ZB)], sem[p]).wait()

    f = pl.kernel(body, out_type=out_type, mesh=mesh,
                  scratch_types=tuple(scratch),
                  compiler_params=pltpu.CompilerParams(
                      use_tc_tiling_on_sc=False))
    zrow = jnp.zeros((ZB, _D), jnp.float32)
    if with_cnt:
        z16 = jnp.zeros((ZB, _CW), jnp.float32)
        ones = jnp.ones((B, _CW), jnp.float32)
        return f(x, gidx, seg, zrow, z16, ones)
    return f(x, gidx, seg, zrow)


def _dense_update(acc2, cnt2, W, b, g, bt, BLK=512):
    SP = acc2.shape[1]

    def body(a_r, c_r, w_r, b_r, g_r, t_r, o_r):
        a = a_r[0] + a_r[1]
        c = c_r[0, :, 0:1] + c_r[1, :, 0:1]
        m = a / jnp.maximum(c, 1.0)
        z = jnp.dot(m, w_r[...], preferred_element_type=jnp.float32) + b_r[...]
        mu = jnp.mean(z, axis=-1, keepdims=True)
        var = jnp.mean((z - mu) ** 2, axis=-1, keepdims=True)
        y = (z - mu) * lax.rsqrt(var + 1e-5) * g_r[...] + t_r[...]
        o_r[...] = jnp.maximum(y, 0.0)

    return pl.pallas_call(
        body,
        grid=(SP // BLK,),
        in_specs=[
            pl.BlockSpec((_NC, BLK, _D), lambda i: (0, i, 0)),
            pl.BlockSpec((_NC, BLK, _CW), lambda i: (0, i, 0)),
            pl.BlockSpec((_D, _D), lambda i: (0, 0)),
            pl.BlockSpec((1, _D), lambda i: (0, 0)),
            pl.BlockSpec((1, _D), lambda i: (0, 0)),
            pl.BlockSpec((1, _D), lambda i: (0, 0)),
        ],
        out_specs=pl.BlockSpec((BLK, _D), lambda i: (i, 0)),
        out_shape=jax.ShapeDtypeStruct((SP, _D), jnp.float32),
    )(acc2, cnt2, W, b.reshape(1, _D), g.reshape(1, _D), bt.reshape(1, _D))


def _final_update(acc2, cnt2, W, b, g, bt, Wc, bc, BLK=512):
    SP = acc2.shape[1]
    Wcp = jnp.zeros((_D, _D), jnp.float32).at[:, :_C].set(Wc)
    bcp = jnp.full((1, _D), -1e30, jnp.float32).at[0, :_C].set(bc)

    def body(a_r, c_r, w_r, b_r, g_r, t_r, wc_r, bc_r, o_r):
        a = a_r[0] + a_r[1]
        c = c_r[0, :, 0:1] + c_r[1, :, 0:1]
        m = a / jnp.maximum(c, 1.0)
        z = jnp.dot(m, w_r[...], preferred_element_type=jnp.float32) + b_r[...]
        mu = jnp.mean(z, axis=-1, keepdims=True)
        var = jnp.mean((z - mu) ** 2, axis=-1, keepdims=True)
        h = jnp.maximum((z - mu) * lax.rsqrt(var + 1e-5) * g_r[...] + t_r[...],
                        0.0)
        lg = jnp.dot(h, wc_r[...], preferred_element_type=jnp.float32) + bc_r[...]
        mx = jnp.max(lg, axis=-1, keepdims=True)
        lse = mx + jnp.log(jnp.sum(jnp.exp(lg - mx), axis=-1, keepdims=True))
        o_r[...] = lg - lse

    return pl.pallas_call(
        body,
        grid=(SP // BLK,),
        in_specs=[
            pl.BlockSpec((_NC, BLK, _D), lambda i: (0, i, 0)),
            pl.BlockSpec((_NC, BLK, _CW), lambda i: (0, i, 0)),
            pl.BlockSpec((_D, _D), lambda i: (0, 0)),
            pl.BlockSpec((1, _D), lambda i: (0, 0)),
            pl.BlockSpec((1, _D), lambda i: (0, 0)),
            pl.BlockSpec((1, _D), lambda i: (0, 0)),
            pl.BlockSpec((_D, _D), lambda i: (0, 0)),
            pl.BlockSpec((1, _D), lambda i: (0, 0)),
        ],
        out_specs=pl.BlockSpec((BLK, _D), lambda i: (i, 0)),
        out_shape=jax.ShapeDtypeStruct((SP, _D), jnp.float32),
    )(acc2, cnt2, W, b.reshape(1, _D), g.reshape(1, _D), bt.reshape(1, _D),
      Wcp, bcp)


def kernel(node_x, nodes_map, edge_batch, edges_map, node_batch,
           We, be, gE, bE, Wn, bn, gN, bN, Wc, bc):
    nm = nodes_map.astype(jnp.int32)
    em = edges_map.astype(jnp.int32)
    eb = edge_batch.astype(jnp.int32)
    nb = node_batch.astype(jnp.int32)

    acc_e, cnt_e = _seg_sum(node_x, nm, eb, _MP, True, 200)
    edge_x = _dense_update(acc_e, cnt_e, We[0], be[0], gE[0], bE[0])
    acc_n, cnt_n = _seg_sum(edge_x, em, nb, _NP, True, 80)
    x1 = _dense_update(acc_n, cnt_n, Wn[0], bn[0], gN[0], bN[0])
    acc_e2 = _seg_sum(x1, nm, eb, _MP, False, 200)
    edge_x2 = _dense_update(acc_e2, cnt_e, We[1], be[1], gE[1], bE[1])
    acc_n2 = _seg_sum(edge_x2, em, nb, _NP, False, 80)
    out = _final_update(acc_n2, cnt_n, Wn[1], bn[1], gN[1], bN[1], Wc, bc)
    return out[:_N, :_C]


# speedup vs baseline: 8.1954x; 1.0000x over previous
import jax
import jax.numpy as jnp
from jax import lax
from jax.experimental import pallas as pl
from jax.experimental.pallas import tpu as pltpu
from jax.experimental.pallas import tpu_sc as plsc

_N = 10000
_M = 5000
_I = 320000
_D = 128
_C = 40
_NC = 2
_NS = 16
_NW = _NC * _NS
_CH = _I // _NW
_MP = 5120
_NP = 10240
_CW = 16


def _seg_sum(x, gidx, seg, SP, with_cnt, B):
    rpt = SP // _NS
    nchunk = _CH // B
    ZB = 80
    mesh = plsc.VectorSubcoreMesh(core_axis_name="c", subcore_axis_name="s",
                                  num_cores=_NC, num_subcores=_NS)

    if with_cnt:
        out_type = (
            jax.ShapeDtypeStruct((_NC, SP, _D), jnp.float32),
            jax.ShapeDtypeStruct((_NC, SP, _CW), jnp.float32),
        )
    else:
        out_type = jax.ShapeDtypeStruct((_NC, SP, _D), jnp.float32)

    scratch = [
        pltpu.VMEM_SHARED((SP, _D), jnp.float32),
        pltpu.VMEM((B, _D), jnp.float32),
        pltpu.VMEM((B, _D), jnp.float32),
        pltpu.VMEM((B,), jnp.int32),
        pltpu.VMEM((B,), jnp.int32),
        pltpu.VMEM((B,), jnp.int32),
        pltpu.VMEM((B,), jnp.int32),
        pltpu.SemaphoreType.DMA,
        pltpu.SemaphoreType.DMA,
        pltpu.SemaphoreType.DMA,
        pltpu.SemaphoreType.DMA,
    ]
    if with_cnt:
        scratch.append(pltpu.VMEM_SHARED((SP, _CW), jnp.float32))
        scratch.append(pltpu.VMEM((B, _CW), jnp.float32))
        scratch.append(pltpu.VMEM((ZB, _CW), jnp.float32))

    def body(x_h, gi_h, sg_h, zr_h, *rest):
        if with_cnt:
            (z16_h, on_h, acc_o, cnt_o, acc_s,
             rows0, rows1, gi0, gi1, sg0, sg1, sem0, sem1, ssem0, ssem1,
             cnt_s, ones_v, zc_v) = rest
        else:
            (acc_o, acc_s, rows0, rows1, gi0, gi1, sg0, sg1,
             sem0, sem1, ssem0, ssem1) = rest
        rows = (rows0, rows1)
        gi = (gi0, gi1)
        sg = (sg0, sg1)
        sem = (sem0, sem1)
        ssem = (ssem0, ssem1)
        cid = lax.axis_index("c")
        sid = lax.axis_index("s")
        wid = cid * _NS + sid
        r0 = sid * rpt
        base0 = wid * _CH

        def load_idx(n, q):
            b = base0 + n * B
            pltpu.sync_copy(gi_h.at[pl.ds(b, B)], gi[q])
            pltpu.sync_copy(sg_h.at[pl.ds(b, B)], sg[q])

        def start_gather(q):
            pltpu.async_copy(x_h.at[gi[q]], rows[q], sem[q])

        def wait_gather(p):
            pltpu.make_async_copy(x_h.at[gi[p]], rows[p], sem[p]).wait()

        def start_scatter(p):
            pltpu.async_copy(rows[p], acc_s.at[sg[p]], ssem[p], add=True)
            if with_cnt:
                pltpu.async_copy(ones_v, cnt_s.at[sg[p]], ssem[p], add=True)

        def wait_scatter(p):
            pltpu.make_async_copy(rows[p], acc_s.at[sg[p]], ssem[p]).wait()
            if with_cnt:
                pltpu.make_async_copy(ones_v, cnt_s.at[sg[p]],
                                      ssem[p]).wait()

        load_idx(0, 0)
        start_gather(0)
        pltpu.sync_copy(zr_h, rows1.at[pl.ds(0, ZB)])
        if with_cnt:
            pltpu.sync_copy(z16_h, zc_v)
            pltpu.sync_copy(on_h, ones_v)
        for j in range(rpt // ZB):
            pltpu.sync_copy(rows1.at[pl.ds(0, ZB)],
                            acc_s.at[pl.ds(r0 + j * ZB, ZB)])
            if with_cnt:
                pltpu.sync_copy(zc_v, cnt_s.at[pl.ds(r0 + j * ZB, ZB)])
        plsc.subcore_barrier()

        def half(n, p, first=False):
            q = 1 - p
            if not first:
                wait_scatter(q)
            load_idx(n + 1, q)
            wait_gather(p)
            start_gather(q)
            start_scatter(p)

        npairs = (nchunk - 1) // 2
        half(0, 0, first=True)
        half(1, 1)

        def pair(j, carry):
            half(2 * j, 0)
            half(2 * j + 1, 1)
            return carry

        lax.fori_loop(1, npairs, pair, 0)
        for n in range(2 * npairs, nchunk):
            p = n % 2
            if n + 1 < nchunk:
                wait_scatter(1 - p)
                load_idx(n + 1, 1 - p)
            wait_gather(p)
            if n + 1 < nchunk:
                start_gather(1 - p)
            start_scatter(p)
        wait_scatter(0)
        wait_scatter(1)
        plsc.subcore_barrier()
        nwb = rpt // ZB
        for j in range(nwb):
            p = j % 2
            if j >= 2:
                pltpu.make_async_copy(
                    rows[p].at[pl.ds(0, ZB)],
                    acc_o.at[cid, pl.ds(r0 + (j - 2) * ZB, ZB)],
                    sem[p]).wait()
            pltpu.sync_copy(acc_s.at[pl.ds(r0 + j * ZB, ZB)],
                            rows[p].at[pl.ds(0, ZB)])
            pltpu.async_copy(rows[p].at[pl.ds(0, ZB)],
                             acc_o.at[cid, pl.ds(r0 + j * ZB, ZB)], sem[p])
        for j in range(max(nwb - 2, 0), nwb):
            p = j % 2
            pltpu.make_async_copy(
                rows[p].at[pl.ds(0, ZB)],
                acc_o.at[cid, pl.ds(r0 + j * ZB, ZB)], sem[p]).wait()
        if with_cnt:
            cbuf = (zc_v, ones_v)
            for j in range(nwb):
                p = j % 2
                if j >= 2:
                    pltpu.make_async_copy(
                        cbuf[p].at[pl.ds(0, ZB)],
                        cnt_o.at[cid, pl.ds(r0 + (j - 2) * ZB, ZB)],
                        sem[p]).wait()
                pltpu.sync_copy(cnt_s.at[pl.ds(r0 + j * ZB, ZB)],
                                cbuf[p].at[pl.ds(0, ZB)])
                pltpu.async_copy(cbuf[p].at[pl.ds(0, ZB)],
                                 cnt_o.at[cid, pl.ds(r0 + j * ZB, ZB)],
                                 sem[p])
            for j in range(max(nwb - 2, 0), nwb):
                p = j % 2
                pltpu.make_async_copy(
                    cbuf[p].at[pl.ds(0, ZB)],
                    cnt_o.at[cid, pl.ds(r0 + j * ZB, ZB)], sem[p]).wait()

    f = pl.kernel(body, out_type=out_type, mesh=mesh,
                  scratch_types=tuple(scratch),
                  compiler_params=pltpu.CompilerParams(
                      use_tc_tiling_on_sc=False))
    zrow = jnp.zeros((ZB, _D), jnp.float32)
    if with_cnt:
        z16 = jnp.zeros((ZB, _CW), jnp.float32)
        ones = jnp.ones((B, _CW), jnp.float32)
        return f(x, gidx, seg, zrow, z16, ones)
    return f(x, gidx, seg, zrow)


def _dense_update(acc2, cnt2, W, b, g, bt, BLK=512):
    SP = acc2.shape[1]

    def body(a_r, c_r, w_r, b_r, g_r, t_r, o_r):
        a = a_r[0] + a_r[1]
        c = c_r[0, :, 0:1] + c_r[1, :, 0:1]
        m = a / jnp.maximum(c, 1.0)
        z = jnp.dot(m, w_r[...], preferred_element_type=jnp.float32) + b_r[...]
        mu = jnp.mean(z, axis=-1, keepdims=True)
        var = jnp.mean((z - mu) ** 2, axis=-1, keepdims=True)
        y = (z - mu) * lax.rsqrt(var + 1e-5) * g_r[...] + t_r[...]
        o_r[...] = jnp.maximum(y, 0.0)

    return pl.pallas_call(
        body,
        grid=(SP // BLK,),
        in_specs=[
            pl.BlockSpec((_NC, BLK, _D), lambda i: (0, i, 0)),
            pl.BlockSpec((_NC, BLK, _CW), lambda i: (0, i, 0)),
            pl.BlockSpec((_D, _D), lambda i: (0, 0)),
            pl.BlockSpec((1, _D), lambda i: (0, 0)),
            pl.BlockSpec((1, _D), lambda i: (0, 0)),
            pl.BlockSpec((1, _D), lambda i: (0, 0)),
        ],
        out_specs=pl.BlockSpec((BLK, _D), lambda i: (i, 0)),
        out_shape=jax.ShapeDtypeStruct((SP, _D), jnp.float32),
    )(acc2, cnt2, W, b.reshape(1, _D), g.reshape(1, _D), bt.reshape(1, _D))


def _final_update(acc2, cnt2, W, b, g, bt, Wc, bc, BLK=512):
    SP = acc2.shape[1]
    Wcp = jnp.zeros((_D, _D), jnp.float32).at[:, :_C].set(Wc)
    bcp = jnp.full((1, _D), -1e30, jnp.float32).at[0, :_C].set(bc)

    def body(a_r, c_r, w_r, b_r, g_r, t_r, wc_r, bc_r, o_r):
        a = a_r[0] + a_r[1]
        c = c_r[0, :, 0:1] + c_r[1, :, 0:1]
        m = a / jnp.maximum(c, 1.0)
        z = jnp.dot(m, w_r[...], preferred_element_type=jnp.float32) + b_r[...]
        mu = jnp.mean(z, axis=-1, keepdims=True)
        var = jnp.mean((z - mu) ** 2, axis=-1, keepdims=True)
        h = jnp.maximum((z - mu) * lax.rsqrt(var + 1e-5) * g_r[...] + t_r[...],
                        0.0)
        lg = jnp.dot(h, wc_r[...], preferred_element_type=jnp.float32) + bc_r[...]
        mx = jnp.max(lg, axis=-1, keepdims=True)
        lse = mx + jnp.log(jnp.sum(jnp.exp(lg - mx), axis=-1, keepdims=True))
        o_r[...] = lg - lse

    return pl.pallas_call(
        body,
        grid=(SP // BLK,),
        in_specs=[
            pl.BlockSpec((_NC, BLK, _D), lambda i: (0, i, 0)),
            pl.BlockSpec((_NC, BLK, _CW), lambda i: (0, i, 0)),
            pl.BlockSpec((_D, _D), lambda i: (0, 0)),
            pl.BlockSpec((1, _D), lambda i: (0, 0)),
            pl.BlockSpec((1, _D), lambda i: (0, 0)),
            pl.BlockSpec((1, _D), lambda i: (0, 0)),
            pl.BlockSpec((_D, _D), lambda i: (0, 0)),
            pl.BlockSpec((1, _D), lambda i: (0, 0)),
        ],
        out_specs=pl.BlockSpec((BLK, _D), lambda i: (i, 0)),
        out_shape=jax.ShapeDtypeStruct((SP, _D), jnp.float32),
    )(acc2, cnt2, W, b.reshape(1, _D), g.reshape(1, _D), bt.reshape(1, _D),
      Wcp, bcp)


def kernel(node_x, nodes_map, edge_batch, edges_map, node_batch,
           We, be, gE, bE, Wn, bn, gN, bN, Wc, bc):
    nm = nodes_map.astype(jnp.int32)
    em = edges_map.astype(jnp.int32)
    eb = edge_batch.astype(jnp.int32)
    nb = node_batch.astype(jnp.int32)

    acc_e, cnt_e = _seg_sum(node_x, nm, eb, _MP, True, 200)
    edge_x = _dense_update(acc_e, cnt_e, We[0], be[0], gE[0], bE[0])
    acc_n, cnt_n = _seg_sum(edge_x, em, nb, _NP, True, 80)
    x1 = _dense_update(acc_n, cnt_n, Wn[0], bn[0], gN[0], bN[0])
    acc_e2 = _seg_sum(x1, nm, eb, _MP, False, 200)
    edge_x2 = _dense_update(acc_e2, cnt_e, We[1], be[1], gE[1], bE[1])
    acc_n2 = _seg_sum(edge_x2, em, nb, _NP, False, 80)
    out = _final_update(acc_n2, cnt_n, Wn[1], bn[1], gN[1], bN[1], Wc, bc)
    return out[:_N, :_C]
